# baseline (speedup 1.0000x reference)
import functools

import jax
import jax.numpy as jnp
from jax import lax
from jax.experimental import pallas as pl
from jax.experimental.pallas import tpu as pltpu
from jax.experimental.pallas import tpu_sc as plsc

_B, _H, _W = 16, 64, 64
_DIM = 384
_DM = 2 * _DIM
_NWORKERS = 32
_ROWS = (_H * _W) // _NWORKERS
_HPW = _ROWS // _W
_LANES = 16


def _sc_body(pe_x_hbm, pe_y_hbm, out_hbm, pe_x_v, pe_y_v, chunk, sem, sem_t):
    c = lax.axis_index("c")
    s = lax.axis_index("s")
    wid = s * 2 + c
    row0 = wid * _ROWS
    h0 = wid * _HPW

    cx = pltpu.async_copy(pe_x_hbm, pe_x_v, sem_t)
    cy = pltpu.async_copy(pe_y_hbm.at[pl.ds(h0, _HPW)], pe_y_v, sem_t)
    cx.wait()
    cy.wait()

    vys = [[pe_y_v[hl, pl.ds(j * _LANES, _LANES)]
            for j in range(_DIM // _LANES)] for hl in range(_HPW)]

    copies = []
    for r0, n in ((0, 8), (8, 56), (64, 64)):
        def fill(w, carry, r0=r0):
            base = r0 + w
            wx = base & (_W - 1)
            in_h0 = base < _W
            for j in range(_DIM // _LANES):
                chunk[base, pl.ds(j * _LANES, _LANES)] = (
                    pe_x_v[wx, pl.ds(j * _LANES, _LANES)])
                chunk[base, pl.ds(_DIM + j * _LANES, _LANES)] = jnp.where(
                    in_h0, vys[0][j], vys[1][j])
            return carry

        lax.fori_loop(0, n, fill, 0)
        for b in range(_B):
            copies.append(pltpu.async_copy(
                chunk.at[pl.ds(r0, n)],
                out_hbm.at[b].at[pl.ds(row0 + r0, n)],
                sem))
    for cp in copies:
        cp.wait()


def kernel(x, pe_x, pe_y):
    bsize, _, h, w = x.shape
    mesh = plsc.VectorSubcoreMesh(core_axis_name="c", subcore_axis_name="s")
    k = functools.partial(
        pl.kernel,
        mesh=mesh,
        out_type=jax.ShapeDtypeStruct((bsize, h * w, _DM), jnp.float32),
        scratch_types=[
            pltpu.VMEM((_W, _DIM), jnp.float32),
            pltpu.VMEM((_HPW, _DIM), jnp.float32),
            pltpu.VMEM((_ROWS, _DM), jnp.float32),
            pltpu.SemaphoreType.DMA,
            pltpu.SemaphoreType.DMA,
        ],
    )(_sc_body)
    return k(pe_x, pe_y)


# speedup vs baseline: 1.0150x; 1.0150x over previous
import functools

import jax
import jax.numpy as jnp
from jax import lax
from jax.experimental import pallas as pl
from jax.experimental.pallas import tpu as pltpu
from jax.experimental.pallas import tpu_sc as plsc

_B, _H, _W = 16, 64, 64
_DIM = 384
_DM = 2 * _DIM
_NWORKERS = 32
_ROWS = (_H * _W) // _NWORKERS
_HPW = _ROWS // _W
_LANES = 16


def _sc_body(pe_x_hbm, pe_y_hbm, out_hbm, pe_x_v, pe_y_v, chunk, sem, sem_t):
    c = lax.axis_index("c")
    s = lax.axis_index("s")
    wid = s * 2 + c
    row0 = wid * _ROWS
    h0 = wid * _HPW

    cx = pltpu.async_copy(pe_x_hbm, pe_x_v, sem_t)
    cy = pltpu.async_copy(pe_y_hbm.at[pl.ds(h0, _HPW)], pe_y_v, sem_t)
    cx.wait()
    cy.wait()

    copies = []
    for r0, n, hl in ((0, 16, 0), (16, 48, 0), (64, 64, 1)):
        vys = [pe_y_v[hl, pl.ds(j * _LANES, _LANES)]
               for j in range(_DIM // _LANES)]
        wo = r0 % _W

        def fill(w, carry, r0=r0, wo=wo, vys=vys):
            base = r0 + w
            for j in range(_DIM // _LANES):
                chunk[base, pl.ds(j * _LANES, _LANES)] = (
                    pe_x_v[wo + w, pl.ds(j * _LANES, _LANES)])
                chunk[base, pl.ds(_DIM + j * _LANES, _LANES)] = vys[j]
            return carry

        lax.fori_loop(0, n, fill, 0)
        for b in range(_B):
            bb = (b + wid) & (_B - 1)
            copies.append(pltpu.async_copy(
                chunk.at[pl.ds(r0, n)],
                out_hbm.at[bb].at[pl.ds(row0 + r0, n)],
                sem))
    for cp in copies:
        cp.wait()


def kernel(x, pe_x, pe_y):
    bsize, _, h, w = x.shape
    mesh = plsc.VectorSubcoreMesh(core_axis_name="c", subcore_axis_name="s")
    k = functools.partial(
        pl.kernel,
        mesh=mesh,
        out_type=jax.ShapeDtypeStruct((bsize, h * w, _DM), jnp.float32),
        scratch_types=[
            pltpu.VMEM((_W, _DIM), jnp.float32),
            pltpu.VMEM((_HPW, _DIM), jnp.float32),
            pltpu.VMEM((_ROWS, _DM), jnp.float32),
            pltpu.SemaphoreType.DMA,
            pltpu.SemaphoreType.DMA,
        ],
    )(_sc_body)
    return k(pe_x, pe_y)


# speedup vs baseline: 1.0211x; 1.0060x over previous
import functools

import jax
import jax.numpy as jnp
from jax import lax
from jax.experimental import pallas as pl
from jax.experimental.pallas import tpu as pltpu
from jax.experimental.pallas import tpu_sc as plsc

_B, _H, _W = 16, 64, 64
_DIM = 384
_DM = 2 * _DIM
_NWORKERS = 32
_ROWS = (_H * _W) // _NWORKERS
_HPW = _ROWS // _W
_LANES = 16


def _sc_body(pe_x_hbm, pe_y_hbm, out_hbm, pe_x_v, pe_y_v, chunk, sem, sem_t):
    c = lax.axis_index("c")
    s = lax.axis_index("s")
    wid = s * 2 + c
    row0 = wid * _ROWS
    h0 = wid * _HPW

    cx = pltpu.async_copy(pe_x_hbm, pe_x_v, sem_t)
    cy = pltpu.async_copy(pe_y_hbm.at[pl.ds(h0, _HPW)], pe_y_v, sem_t)
    cx.wait()
    cy.wait()

    copies = []
    for r0, n, hl in ((0, 16, 0), (16, 48, 0), (64, 64, 1)):
        vys = [pe_y_v[hl, pl.ds(j * _LANES, _LANES)]
               for j in range(_DIM // _LANES)]
        wo = r0 % _W

        def fill(w, carry, r0=r0, wo=wo, vys=vys):
            base = r0 + w
            for j in range(_DIM // _LANES):
                chunk[base, pl.ds(j * _LANES, _LANES)] = (
                    pe_x_v[wo + w, pl.ds(j * _LANES, _LANES)])
                chunk[base, pl.ds(_DIM + j * _LANES, _LANES)] = vys[j]
            return carry

        lax.fori_loop(0, n, fill, 0)
        for b in range(_B):
            copies.append(pltpu.async_copy(
                chunk.at[pl.ds(r0, n)],
                out_hbm.at[b].at[pl.ds(row0 + r0, n)],
                sem))
    for cp in copies:
        cp.wait()


def kernel(x, pe_x, pe_y):
    bsize, _, h, w = x.shape
    mesh = plsc.VectorSubcoreMesh(core_axis_name="c", subcore_axis_name="s")
    k = functools.partial(
        pl.kernel,
        mesh=mesh,
        out_type=jax.ShapeDtypeStruct((bsize, h * w, _DM), jnp.float32),
        scratch_types=[
            pltpu.VMEM((_W, _DIM), jnp.float32),
            pltpu.VMEM((_HPW, _DIM), jnp.float32),
            pltpu.VMEM((_ROWS, _DM), jnp.float32),
            pltpu.SemaphoreType.DMA,
            pltpu.SemaphoreType.DMA,
        ],
    )(_sc_body)
    return k(pe_x, pe_y)
